# Initial kernel scaffold; baseline (speedup 1.0000x reference)
import jax
import jax.numpy as jnp
from jax.experimental import pallas as pl


def kernel(x, edge_index, edge_attr, W0, root0, b0, W1, root1, b1, W2, root2, b2):
    raise NotImplementedError("write your pallas kernel here")


# speedup vs baseline: 8.0995x; 8.0995x over previous
import functools

import jax
import jax.numpy as jnp
from jax import lax
from jax.experimental import pallas as pl
from jax.experimental.pallas import tpu as pltpu
from jax.experimental.pallas import tpu_sc as plsc

N = 10000
E = 640000
KS = 5
K = KS ** 3
C = 64

NC = 2
NS = 16
NW = NC * NS
CH = 128
NCH_TOT = E // CH
CHUNKS_W = NCH_TOT // NW
CHUNKS_EXTRA = NCH_TOT - CHUNKS_W * NW

ROWS_T = 624


def _prep_body(attr_ref, src_ref, gidx_ref, bw_ref):
    fr = []
    fl = []
    for d in range(3):
        pos = jnp.clip(attr_ref[d:d + 1, :], 0.0, 1.0) * (KS - 1.0)
        f = jnp.clip(jnp.floor(pos), 0.0, KS - 2.0)
        fr.append(pos - f)
        fl.append(f.astype(jnp.int32))
    src = src_ref[0:1, :]
    for s in range(8):
        b = None
        for d in range(3):
            bit = (s >> d) & 1
            bd = fr[d] if bit else (1.0 - fr[d])
            b = bd if b is None else b * bd
        bw_ref[s:s + 1, :] = b
    for p in range(4):
        bit1 = p & 1
        bit2 = (p >> 1) & 1
        k12 = (fl[1] + bit1) + (fl[2] + bit2) * KS
        gidx_ref[p:p + 1, :] = (k12 * 4 + fl[0]) * N + src


def _edge_prep(attr_t, src2):
    BE = 6400
    grid = E // BE
    return pl.pallas_call(
        _prep_body,
        grid=(grid,),
        in_specs=[
            pl.BlockSpec((3, BE), lambda i: (0, i)),
            pl.BlockSpec((1, BE), lambda i: (0, i)),
        ],
        out_specs=[
            pl.BlockSpec((4, BE), lambda i: (0, i)),
            pl.BlockSpec((8, BE), lambda i: (0, i)),
        ],
        out_shape=[
            jax.ShapeDtypeStruct((4, E), jnp.int32),
            jax.ShapeDtypeStruct((8, E), jnp.float32),
        ],
    )(attr_t, src2)


BN = 400
GRID_N = N // BN


def _h_body(agg_ref, degp_ref, rp_ref, out_ref):
    deg = jnp.sum(degp_ref[:, :, C:], axis=(0, 2)) * (1.0 / C)
    inv = 1.0 / jnp.maximum(deg, 1.0)
    agg = agg_ref[0, :, :C] + agg_ref[1, :, :C]
    out_ref[...] = jnp.maximum(agg * inv[:, None] + rp_ref[...], 0.0)


def _h_step(agg, degp, r_prev):
    return pl.pallas_call(
        _h_body,
        grid=(GRID_N,),
        in_specs=[
            pl.BlockSpec((NC, BN, 2 * C), lambda i: (0, i, 0)),
            pl.BlockSpec((NC, BN, 2 * C), lambda i: (0, i, 0)),
            pl.BlockSpec((BN, C), lambda i: (i, 0)),
        ],
        out_specs=pl.BlockSpec((BN, C), lambda i: (i, 0)),
        out_shape=jax.ShapeDtypeStruct((N, C), jnp.float32),
    )(agg, degp, r_prev)


NPAIR = 100


def _z_body(h_ref, wa_ref, wb_ref, root_ref, bias_ref, z_ref, r_ref):
    j = pl.program_id(0)
    h = h_ref[...]
    za = jnp.dot(h, wa_ref[0], preferred_element_type=jnp.float32)
    zb = jnp.dot(h, wb_ref[0], preferred_element_type=jnp.float32)
    z_ref[...] = jnp.concatenate([za, zb], axis=1)

    @pl.when(j == 0)
    def _():
        r_ref[...] = (
            jnp.dot(h, root_ref[...], preferred_element_type=jnp.float32)
            + bias_ref[...]
        )


def _z_all(h, w, root, bias):
    cin = h.shape[1]
    return pl.pallas_call(
        _z_body,
        grid=(NPAIR,),
        in_specs=[
            pl.BlockSpec((N, cin), lambda j: (0, 0)),
            pl.BlockSpec((1, cin, C), lambda j: (j // 4 * KS + j % 4, 0, 0)),
            pl.BlockSpec((1, cin, C), lambda j: (j // 4 * KS + j % 4 + 1, 0, 0)),
            pl.BlockSpec((cin, C), lambda j: (0, 0)),
            pl.BlockSpec((1, C), lambda j: (0, 0)),
        ],
        out_specs=[
            pl.BlockSpec((N, 2 * C), lambda j: (j, 0)),
            pl.BlockSpec((N, C), lambda j: (0, 0)),
        ],
        out_shape=[
            jax.ShapeDtypeStruct((NPAIR * N, 2 * C), jnp.float32),
            jax.ShapeDtypeStruct((N, C), jnp.float32),
        ],
    )(h, w, w, root, bias)


HC = 64
ZROWS = 64


def _agg_body(with_deg, ztab, gidx, bw, dst2, agg_out,
              acc_sh, gidx_v, bw_v, dst_v, rows_v, m_v, sem):
    cid = lax.axis_index("c")
    sid = lax.axis_index("s")
    wid = sid * NC + cid

    zvec = jnp.zeros((16,), jnp.float32)
    onevec = jnp.ones((16,), jnp.float32)

    def fill_row(e, _):
        for c4 in range(2 * C // 16):
            m_v[e, pl.ds(c4 * 16, 16)] = zvec
        return 0

    lax.fori_loop(0, HC, fill_row, 0)

    row0 = sid * ROWS_T
    nrows = jnp.where(sid == NS - 1, N - (NS - 1) * ROWS_T, ROWS_T)
    nfull = nrows // ZROWS
    rem = ROWS_T - (ROWS_T // ZROWS) * ZROWS

    def zchunk(j, _):
        pltpu.sync_copy(m_v, acc_sh.at[pl.ds(row0 + j * ZROWS, ZROWS)])
        return 0

    lax.fori_loop(0, nfull, zchunk, 0)

    @pl.when(sid != NS - 1)
    def _():
        sl = pl.ds(row0 + (ROWS_T // ZROWS) * ZROWS, rem)
        pltpu.sync_copy(m_v.at[pl.ds(0, rem)], acc_sh.at[sl])

    if with_deg:
        def one_row(e, _):
            for c4 in range(C // 16):
                m_v[e, pl.ds(C + c4 * 16, 16)] = onevec
            return 0

        lax.fori_loop(0, HC, one_row, 0)

    plsc.subcore_barrier()

    nch = jnp.where(wid < CHUNKS_EXTRA, CHUNKS_W + 1, CHUNKS_W)

    def chunk(j, _):
        c = wid + j * NW
        base = c * CH
        pltpu.sync_copy(gidx.at[:, pl.ds(base, CH)], gidx_v)
        pltpu.sync_copy(bw.at[:, pl.ds(base, CH)], bw_v)
        pltpu.sync_copy(dst2.at[pl.ds(c * (CH // HC), CH // HC)], dst_v)
        for half in range(CH // HC):
            cps = [pltpu.async_copy(
                ztab.at[gidx_v.at[p, pl.ds(half * HC, HC)]],
                rows_v.at[p], sem) for p in range(4)]
            for cp in cps:
                cp.wait()

            def group(g, _):
                e0 = g * 16
                bvec = [bw_v[s, pl.ds(half * HC + e0, 16)] for s in range(8)]
                for l in range(16):
                    lane = jnp.full((16,), l, jnp.int32)
                    bsp = [jnp.take_along_axis(bvec[s], lane, axis=0)
                           for s in range(8)]
                    e = e0 + l
                    for c4 in range(C // 16):
                        acc = bsp[0] * rows_v[0, e, pl.ds(c4 * 16, 16)]
                        acc = acc + bsp[1] * rows_v[0, e, pl.ds(C + c4 * 16, 16)]
                        for p in range(1, 4):
                            acc = acc + bsp[2 * p] * rows_v[p, e, pl.ds(c4 * 16, 16)]
                            acc = acc + bsp[2 * p + 1] * rows_v[p, e, pl.ds(C + c4 * 16, 16)]
                        m_v[e, pl.ds(c4 * 16, 16)] = acc
                return 0

            lax.fori_loop(0, HC // 16, group, 0)

            pltpu.sync_copy(m_v, acc_sh.at[dst_v.at[half]], add=True)
        return 0

    lax.fori_loop(0, nch, chunk, 0)

    plsc.subcore_barrier()

    def out_chunk(j, _):
        sl = pl.ds(row0 + j * ZROWS, ZROWS)
        pltpu.sync_copy(acc_sh.at[sl], agg_out.at[cid, sl])
        return 0

    lax.fori_loop(0, nfull, out_chunk, 0)

    @pl.when(sid != NS - 1)
    def _():
        sl = pl.ds(row0 + (ROWS_T // ZROWS) * ZROWS, rem)
        pltpu.sync_copy(acc_sh.at[sl], agg_out.at[cid, sl])


def _make_agg(with_deg):
    scratch = [
        pltpu.VMEM_SHARED((N, 2 * C), jnp.float32),
        pltpu.VMEM((4, CH), jnp.int32),
        pltpu.VMEM((8, CH), jnp.float32),
        pltpu.VMEM((CH // HC, HC), jnp.int32),
        pltpu.VMEM((4, HC, 2 * C), jnp.float32),
        pltpu.VMEM((HC, 2 * C), jnp.float32),
        pltpu.SemaphoreType.DMA,
    ]
    return pl.kernel(
        functools.partial(_agg_body, with_deg),
        out_type=jax.ShapeDtypeStruct((NC, N, 2 * C), jnp.float32),
        mesh=plsc.VectorSubcoreMesh(core_axis_name="c", subcore_axis_name="s",
                                    num_cores=NC, num_subcores=NS),
        scratch_types=scratch,
    )


def kernel(x, edge_index, edge_attr, W0, root0, b0, W1, root1, b1, W2, root2, b2):
    src2 = edge_index[0].astype(jnp.int32).reshape(1, E)
    dst = edge_index[1].astype(jnp.int32)
    attr_t = edge_attr.T

    gidx, bw = _edge_prep(attr_t, src2)
    dst2 = dst.reshape(E // HC, HC)

    agg_deg = _make_agg(True)
    agg_only = _make_agg(False)

    z0, r0 = _z_all(x, W0, root0, b0.reshape(1, C))
    agg0 = agg_deg(z0, gidx, bw, dst2)

    h1 = _h_step(agg0, agg0, r0)
    z1, r1 = _z_all(h1, W1, root1, b1.reshape(1, C))
    agg1 = agg_only(z1, gidx, bw, dst2)

    h2 = _h_step(agg1, agg0, r1)
    z2, r2 = _z_all(h2, W2, root2, b2.reshape(1, C))
    agg2 = agg_only(z2, gidx, bw, dst2)

    return _h_step(agg2, agg0, r2)


# speedup vs baseline: 9.9696x; 1.2309x over previous
import functools

import jax
import jax.numpy as jnp
from jax import lax
from jax.experimental import pallas as pl
from jax.experimental.pallas import tpu as pltpu
from jax.experimental.pallas import tpu_sc as plsc

N = 10000
E = 640000
KS = 5
K = KS ** 3
C = 64

NC = 2
NS = 16
NW = NC * NS
CH = 128
NCH_TOT = E // CH
CHUNKS_W = NCH_TOT // NW
CHUNKS_EXTRA = NCH_TOT - CHUNKS_W * NW

ROWS_T = 624


def _prep_body(attr_ref, src_ref, gidx_ref, bw_ref):
    fr = []
    fl = []
    for d in range(3):
        pos = jnp.clip(attr_ref[d:d + 1, :], 0.0, 1.0) * (KS - 1.0)
        f = jnp.clip(jnp.floor(pos), 0.0, KS - 2.0)
        fr.append(pos - f)
        fl.append(f.astype(jnp.int32))
    src = src_ref[0:1, :]
    for s in range(8):
        b = None
        for d in range(3):
            bit = (s >> d) & 1
            bd = fr[d] if bit else (1.0 - fr[d])
            b = bd if b is None else b * bd
        bw_ref[s:s + 1, :] = b
    for p in range(4):
        bit1 = p & 1
        bit2 = (p >> 1) & 1
        k12 = (fl[1] + bit1) + (fl[2] + bit2) * KS
        gidx_ref[p:p + 1, :] = (k12 * 4 + fl[0]) * N + src


def _edge_prep(attr_t, src2):
    BE = 6400
    grid = E // BE
    return pl.pallas_call(
        _prep_body,
        grid=(grid,),
        in_specs=[
            pl.BlockSpec((3, BE), lambda i: (0, i)),
            pl.BlockSpec((1, BE), lambda i: (0, i)),
        ],
        out_specs=[
            pl.BlockSpec((4, BE), lambda i: (0, i)),
            pl.BlockSpec((8, BE), lambda i: (0, i)),
        ],
        out_shape=[
            jax.ShapeDtypeStruct((4, E), jnp.int32),
            jax.ShapeDtypeStruct((8, E), jnp.float32),
        ],
    )(attr_t, src2)


BN = 400
GRID_N = N // BN


def _h_body(agg_ref, rp_ref, out_ref):
    deg = jnp.sum(agg_ref[:, :, C:], axis=(0, 2)) * (1.0 / C)
    inv = 1.0 / jnp.maximum(deg, 1.0)
    agg = agg_ref[0, :, :C] + agg_ref[1, :, :C]
    out_ref[...] = jnp.maximum(agg * inv[:, None] + rp_ref[...], 0.0)


def _h_step(agg, r_prev):
    return pl.pallas_call(
        _h_body,
        grid=(GRID_N,),
        in_specs=[
            pl.BlockSpec((NC, BN, 2 * C), lambda i: (0, i, 0)),
            pl.BlockSpec((BN, C), lambda i: (i, 0)),
        ],
        out_specs=pl.BlockSpec((BN, C), lambda i: (i, 0)),
        out_shape=jax.ShapeDtypeStruct((N, C), jnp.float32),
    )(agg, r_prev)


NPAIR = 100


def _z_body(h_ref, wa_ref, wb_ref, root_ref, bias_ref, z_ref, r_ref):
    j = pl.program_id(0)
    h = h_ref[...]
    za = jnp.dot(h, wa_ref[0], preferred_element_type=jnp.float32)
    zb = jnp.dot(h, wb_ref[0], preferred_element_type=jnp.float32)
    z_ref[...] = jnp.concatenate([za, zb], axis=1)

    @pl.when(j == 0)
    def _():
        r_ref[...] = (
            jnp.dot(h, root_ref[...], preferred_element_type=jnp.float32)
            + bias_ref[...]
        )


def _z_all(h, w, root, bias):
    cin = h.shape[1]
    return pl.pallas_call(
        _z_body,
        grid=(NPAIR,),
        in_specs=[
            pl.BlockSpec((N, cin), lambda j: (0, 0)),
            pl.BlockSpec((1, cin, C), lambda j: (j // 4 * KS + j % 4, 0, 0)),
            pl.BlockSpec((1, cin, C), lambda j: (j // 4 * KS + j % 4 + 1, 0, 0)),
            pl.BlockSpec((cin, C), lambda j: (0, 0)),
            pl.BlockSpec((1, C), lambda j: (0, 0)),
        ],
        out_specs=[
            pl.BlockSpec((N, 2 * C), lambda j: (j, 0)),
            pl.BlockSpec((N, C), lambda j: (0, 0)),
        ],
        out_shape=[
            jax.ShapeDtypeStruct((NPAIR * N, 2 * C), jnp.float32),
            jax.ShapeDtypeStruct((N, C), jnp.float32),
        ],
    )(h, w, w, root, bias)


HC = 32
NQ = CH // HC
ZROWS = 32


def _agg_body(ztab, gidx, bw, dst2, agg_out,
              acc_sh, gidx_v, bw_v, dst_v, rows_a, rows_b, m0_v, m1_v,
              sem_a, sem_b, sem_s):
    mw = 2 * C
    cid = lax.axis_index("c")
    sid = lax.axis_index("s")
    wid = sid * NC + cid

    zvec = jnp.zeros((16,), jnp.float32)
    onevec = jnp.ones((16,), jnp.float32)

    def fill_row(e, _):
        for c4 in range(mw // 16):
            m0_v[e, pl.ds(c4 * 16, 16)] = zvec
            m1_v[e, pl.ds(c4 * 16, 16)] = zvec
        return 0

    lax.fori_loop(0, HC, fill_row, 0)

    row0 = sid * ROWS_T
    nrows = jnp.where(sid == NS - 1, N - (NS - 1) * ROWS_T, ROWS_T)
    nfull = nrows // ZROWS
    rem = ROWS_T - (ROWS_T // ZROWS) * ZROWS

    def zchunk(j, _):
        sl = pl.ds(row0 + j * ZROWS, ZROWS)
        pltpu.sync_copy(m0_v, acc_sh.at[sl])
        return 0

    lax.fori_loop(0, nfull, zchunk, 0)

    @pl.when(sid != NS - 1)
    def _():
        sl = pl.ds(row0 + (ROWS_T // ZROWS) * ZROWS, rem)
        pltpu.sync_copy(m0_v.at[pl.ds(0, rem)], acc_sh.at[sl])

    def one_row(e, _):
        for c4 in range(C // 16):
            m0_v[e, pl.ds(C + c4 * 16, 16)] = onevec
            m1_v[e, pl.ds(C + c4 * 16, 16)] = onevec
        return 0

    lax.fori_loop(0, HC, one_row, 0)

    plsc.subcore_barrier()

    nch = jnp.where(wid < CHUNKS_EXTRA, CHUNKS_W + 1, CHUNKS_W)

    def compute_quarter(q, rows_v, m_ref):
        def group(g, _):
            e0 = g * 16
            bvec = [bw_v[s, pl.ds(q * HC + e0, 16)] for s in range(8)]

            def lane_body(l, _):
                lane = jnp.full((16,), l, jnp.int32)
                e = e0 + l
                accs = [None] * (C // 16)
                for p in range(4):
                    b0 = jnp.take_along_axis(bvec[2 * p], lane, axis=0)
                    b1 = jnp.take_along_axis(bvec[2 * p + 1], lane, axis=0)
                    for c4 in range(C // 16):
                        v = (b0 * rows_v[p, e, pl.ds(c4 * 16, 16)]
                             + b1 * rows_v[p, e, pl.ds(C + c4 * 16, 16)])
                        accs[c4] = v if accs[c4] is None else accs[c4] + v
                for c4 in range(C // 16):
                    m_ref[e, pl.ds(c4 * 16, 16)] = accs[c4]
                return 0

            lax.fori_loop(0, 16, lane_body, 0)
            return 0

        lax.fori_loop(0, HC // 16, group, 0)

    def gather_q(q):
        buf = rows_b if q & 1 else rows_a
        sem = sem_b if q & 1 else sem_a
        return [pltpu.async_copy(
            ztab.at[gidx_v.at[p, pl.ds(q * HC, HC)]], buf.at[p], sem)
            for p in range(4)]

    def chunk(j, _):
        c = wid + j * NW
        base = c * CH
        pltpu.sync_copy(gidx.at[:, pl.ds(base, CH)], gidx_v)
        pltpu.sync_copy(bw.at[:, pl.ds(base, CH)], bw_v)
        pltpu.sync_copy(dst2.at[pl.ds(c * NQ, NQ)], dst_v)
        cps = {0: gather_q(0)}
        scat = {}
        for q in range(NQ):
            if q + 1 < NQ:
                cps[q + 1] = gather_q(q + 1)
            for cp in cps[q]:
                cp.wait()
            if q >= 2:
                scat[q - 2].wait()
            m_ref = m1_v if q & 1 else m0_v
            compute_quarter(q, rows_b if q & 1 else rows_a, m_ref)
            scat[q] = pltpu.async_copy(m_ref, acc_sh.at[dst_v.at[q]], sem_s,
                                       add=True)
        scat[NQ - 2].wait()
        scat[NQ - 1].wait()
        return 0

    lax.fori_loop(0, nch, chunk, 0)

    plsc.subcore_barrier()

    def out_chunk(j, _):
        sl = pl.ds(row0 + j * ZROWS, ZROWS)
        pltpu.sync_copy(acc_sh.at[sl], agg_out.at[cid, sl])
        return 0

    lax.fori_loop(0, nfull, out_chunk, 0)

    @pl.when(sid != NS - 1)
    def _():
        sl = pl.ds(row0 + (ROWS_T // ZROWS) * ZROWS, rem)
        pltpu.sync_copy(acc_sh.at[sl], agg_out.at[cid, sl])


def _make_agg():
    scratch = [
        pltpu.VMEM_SHARED((N, 2 * C), jnp.float32),
        pltpu.VMEM((4, CH), jnp.int32),
        pltpu.VMEM((8, CH), jnp.float32),
        pltpu.VMEM((NQ, HC), jnp.int32),
        pltpu.VMEM((4, HC, 2 * C), jnp.float32),
        pltpu.VMEM((4, HC, 2 * C), jnp.float32),
        pltpu.VMEM((HC, 2 * C), jnp.float32),
        pltpu.VMEM((HC, 2 * C), jnp.float32),
        pltpu.SemaphoreType.DMA,
        pltpu.SemaphoreType.DMA,
        pltpu.SemaphoreType.DMA,
    ]
    return pl.kernel(
        _agg_body,
        out_type=jax.ShapeDtypeStruct((NC, N, 2 * C), jnp.float32),
        mesh=plsc.VectorSubcoreMesh(core_axis_name="c", subcore_axis_name="s",
                                    num_cores=NC, num_subcores=NS),
        scratch_types=scratch,
    )


def kernel(x, edge_index, edge_attr, W0, root0, b0, W1, root1, b1, W2, root2, b2):
    src2 = edge_index[0].astype(jnp.int32).reshape(1, E)
    dst = edge_index[1].astype(jnp.int32)
    attr_t = edge_attr.T

    gidx, bw = _edge_prep(attr_t, src2)
    dst2 = dst.reshape(E // HC, HC)

    agg_k = _make_agg()

    z0, r0 = _z_all(x, W0, root0, b0.reshape(1, C))
    agg0 = agg_k(z0, gidx, bw, dst2)

    h1 = _h_step(agg0, r0)
    z1, r1 = _z_all(h1, W1, root1, b1.reshape(1, C))
    agg1 = agg_k(z1, gidx, bw, dst2)

    h2 = _h_step(agg1, r1)
    z2, r2 = _z_all(h2, W2, root2, b2.reshape(1, C))
    agg2 = agg_k(z2, gidx, bw, dst2)

    return _h_step(agg2, r2)
